# baseline (speedup 1.0000x reference)
import functools

import jax
import jax.numpy as jnp
from jax import lax
from jax.experimental import pallas as pl
from jax.experimental.pallas import tpu as pltpu
from jax.experimental.pallas import tpu_sc as plsc


@functools.lru_cache(maxsize=None)
def _make_sc_gather(V, H, B):
    info = plsc.get_sparse_core_info()
    NC, NS, L = info.num_cores, info.num_subcores, info.num_lanes
    NW = NC * NS
    assert H == NW and B % L == 0
    n_grp = B // L
    mesh = plsc.VectorSubcoreMesh(core_axis_name="c", subcore_axis_name="s")

    @functools.partial(
        pl.kernel,
        mesh=mesh,
        out_type=jax.ShapeDtypeStruct((H + 1, B), jnp.float32),
        scratch_types=[
            pltpu.VMEM((B,), jnp.int32),
            pltpu.VMEM((1, V), jnp.float32),
            pltpu.VMEM((1, B), jnp.float32),
            pltpu.SemaphoreType.DMA,
            pltpu.SemaphoreType.DMA,
        ],
        compiler_params=pltpu.CompilerParams(needs_layout_passes=False),
    )
    def gather_kernel(idx_hbm, et_hbm, out_hbm, idx_v, row_v, out_v,
                      sem_i, sem_r):
        wid = lax.axis_index("s") * NC + lax.axis_index("c")
        zero = jnp.full((L,), 0, jnp.int32)
        c_idx = pltpu.async_copy(idx_hbm, idx_v, sem_i)
        c_row = pltpu.async_copy(et_hbm.at[pl.ds(wid, 1)], row_v, sem_r)

        @pl.when(wid == 0)
        def _():
            for g in range(n_grp):
                plsc.store_scatter(
                    out_v, [zero, lax.iota(jnp.int32, L) + g * L],
                    jnp.full((L,), 1.0, jnp.float32))
            pltpu.sync_copy(out_v, out_hbm.at[pl.ds(H, 1)])

        c_idx.wait()
        c_row.wait()
        for g in range(n_grp):
            ids = idx_v[pl.ds(g * L, L)]
            vals = plsc.load_gather(row_v, [zero, ids])
            plsc.store_scatter(
                out_v, [zero, lax.iota(jnp.int32, L) + g * L], vals)
        pltpu.sync_copy(out_v, out_hbm.at[pl.ds(wid, 1)])

    return gather_kernel


def _proj_body(w_ref, b_ref, h_ref, out_ref):
    lhs = jnp.concatenate([w_ref[...], b_ref[...]], axis=0)
    out_ref[...] = lax.dot_general(
        lhs, h_ref[...],
        (((0,), (0,)), ((), ())),
        preferred_element_type=jnp.float32,
    )


@functools.lru_cache(maxsize=None)
def _make_tc_proj(B, H, V, v_tile):
    grid = (pl.cdiv(V, v_tile),)
    return pl.pallas_call(
        _proj_body,
        grid=grid,
        in_specs=[
            pl.BlockSpec((H, v_tile), lambda j: (0, j)),
            pl.BlockSpec((1, v_tile), lambda j: (0, j)),
            pl.BlockSpec((H + 1, B), lambda j: (0, 0)),
        ],
        out_specs=pl.BlockSpec((v_tile, B), lambda j: (j, 0)),
        out_shape=jax.ShapeDtypeStruct((V, B), jnp.float32),
        compiler_params=pltpu.CompilerParams(
            dimension_semantics=("parallel",),
        ),
    )


def kernel(input_ids, embed_table, proj_weight, proj_bias):
    B, = input_ids.shape
    V, H = embed_table.shape
    et_t = jnp.transpose(embed_table)
    w_t = jnp.transpose(proj_weight)
    hidden_aug_t = _make_sc_gather(V, H, B)(input_ids.astype(jnp.int32), et_t)
    logits_t = _make_tc_proj(B, H, V, 3584)(
        w_t, proj_bias.reshape(1, V), hidden_aug_t
    )
    return jnp.transpose(logits_t)


# speedup vs baseline: 1.0034x; 1.0034x over previous
import functools

import jax
import jax.numpy as jnp
from jax import lax
from jax.experimental import pallas as pl
from jax.experimental.pallas import tpu as pltpu
from jax.experimental.pallas import tpu_sc as plsc


@functools.lru_cache(maxsize=None)
def _make_sc_gather(V, H, B):
    info = plsc.get_sparse_core_info()
    NC, NS, L = info.num_cores, info.num_subcores, info.num_lanes
    NW = NC * NS
    assert H == NW and B % L == 0
    n_grp = B // L
    mesh = plsc.VectorSubcoreMesh(core_axis_name="c", subcore_axis_name="s")

    @functools.partial(
        pl.kernel,
        mesh=mesh,
        out_type=jax.ShapeDtypeStruct((H + 1, B), jnp.float32),
        scratch_types=[
            pltpu.VMEM((B,), jnp.int32),
            pltpu.VMEM((1, V), jnp.float32),
            pltpu.VMEM((1, B), jnp.float32),
            pltpu.SemaphoreType.DMA,
            pltpu.SemaphoreType.DMA,
        ],
        compiler_params=pltpu.CompilerParams(needs_layout_passes=False),
    )
    def gather_kernel(idx_hbm, et_hbm, out_hbm, idx_v, row_v, out_v,
                      sem_i, sem_r):
        wid = lax.axis_index("s") * NC + lax.axis_index("c")
        zero = jnp.full((L,), 0, jnp.int32)
        c_idx = pltpu.async_copy(idx_hbm, idx_v, sem_i)
        c_row = pltpu.async_copy(et_hbm.at[pl.ds(wid, 1)], row_v, sem_r)

        @pl.when(wid == 0)
        def _():
            for g in range(n_grp):
                plsc.store_scatter(
                    out_v, [zero, lax.iota(jnp.int32, L) + g * L],
                    jnp.full((L,), 1.0, jnp.float32))
            pltpu.sync_copy(out_v, out_hbm.at[pl.ds(H, 1)])

        c_idx.wait()
        c_row.wait()
        for g in range(n_grp):
            ids = idx_v[pl.ds(g * L, L)]
            vals = plsc.load_gather(row_v, [zero, ids])
            plsc.store_scatter(
                out_v, [zero, lax.iota(jnp.int32, L) + g * L], vals)
        pltpu.sync_copy(out_v, out_hbm.at[pl.ds(wid, 1)])

    return gather_kernel


def _proj_body(w_ref, b_ref, h_ref, out_ref):
    lhs = jnp.concatenate([w_ref[...], b_ref[...]], axis=0)
    out_ref[...] = lax.dot_general(
        lhs, h_ref[...],
        (((0,), (0,)), ((), ())),
        preferred_element_type=jnp.float32,
    )


@functools.lru_cache(maxsize=None)
def _make_tc_proj(B, H, V, v_tile):
    grid = (pl.cdiv(V, v_tile),)
    return pl.pallas_call(
        _proj_body,
        grid=grid,
        in_specs=[
            pl.BlockSpec((H, v_tile), lambda j: (0, j)),
            pl.BlockSpec((1, v_tile), lambda j: (0, j)),
            pl.BlockSpec((H + 1, B), lambda j: (0, 0)),
        ],
        out_specs=pl.BlockSpec((v_tile, B), lambda j: (j, 0)),
        out_shape=jax.ShapeDtypeStruct((V, B), jnp.float32),
        compiler_params=pltpu.CompilerParams(
            dimension_semantics=("parallel",),
        ),
    )


def kernel(input_ids, embed_table, proj_weight, proj_bias):
    B, = input_ids.shape
    V, H = embed_table.shape
    et_t = jnp.transpose(embed_table)
    w_t = jnp.transpose(proj_weight)
    hidden_aug_t = _make_sc_gather(V, H, B)(input_ids.astype(jnp.int32), et_t)
    logits_t = _make_tc_proj(B, H, V, 4096)(
        w_t, proj_bias.reshape(1, V), hidden_aug_t
    )
    return jnp.transpose(logits_t)
